# Initial kernel scaffold; baseline (speedup 1.0000x reference)
import jax
import jax.numpy as jnp
from jax.experimental import pallas as pl


def kernel(x, pos, batch, W, b):
    raise NotImplementedError("write your pallas kernel here")


# speedup vs baseline: 6.9556x; 6.9556x over previous
import functools

import jax
import jax.numpy as jnp
from jax import lax
from jax.experimental import pallas as pl
from jax.experimental.pallas import tpu as pltpu
from jax.experimental.pallas import tpu_sc as plsc

N = 16384
M = 4096
KNN = 17
KPAD = 20
IN_CH = 256
OUT_CH = 512
EPS = 1e-5
BIGI = 2 ** 30


def _fps_body(px_ref, py_ref, pz_ref, idx_ref, qx_ref, qy_ref, qz_ref, dist_ref):
    px = px_ref[...]
    py = py_ref[...]
    pz = pz_ref[...]
    lin = (lax.broadcasted_iota(jnp.int32, (128, 128), 0) * 128
           + lax.broadcasted_iota(jnp.int32, (128, 128), 1))
    olin = (lax.broadcasted_iota(jnp.int32, (32, 128), 0) * 128
            + lax.broadcasted_iota(jnp.int32, (32, 128), 1))

    first = lin == 0
    x0 = jnp.sum(jnp.where(first, px, 0.0))
    y0 = jnp.sum(jnp.where(first, py, 0.0))
    z0 = jnp.sum(jnp.where(first, pz, 0.0))
    w0 = olin == 0
    idx_ref[...] = jnp.zeros((32, 128), jnp.int32)
    qx_ref[...] = jnp.where(w0, x0, 0.0)
    qy_ref[...] = jnp.where(w0, y0, 0.0)
    qz_ref[...] = jnp.where(w0, z0, 0.0)
    dist_ref[...] = jnp.full((128, 128), jnp.inf, jnp.float32)

    def body(i, carry):
        lx, ly, lz = carry
        dx = px - lx
        dy = py - ly
        dz = pz - lz
        d = dx * dx + dy * dy + dz * dz
        dist = jnp.minimum(dist_ref[...], d)
        dist_ref[...] = dist
        m = jnp.max(dist)
        pidx = jnp.min(jnp.where(dist == m, lin, BIGI))
        sel = lin == pidx
        nx = jnp.sum(jnp.where(sel, px, 0.0))
        ny = jnp.sum(jnp.where(sel, py, 0.0))
        nz = jnp.sum(jnp.where(sel, pz, 0.0))
        w = olin == i
        idx_ref[...] = jnp.where(w, pidx, idx_ref[...])
        qx_ref[...] = jnp.where(w, nx, qx_ref[...])
        qy_ref[...] = jnp.where(w, ny, qy_ref[...])
        qz_ref[...] = jnp.where(w, nz, qz_ref[...])
        return (nx, ny, nz)

    lax.fori_loop(1, M, body, (x0, y0, z0))


_fps_call = pl.pallas_call(
    _fps_body,
    out_shape=(
        jax.ShapeDtypeStruct((32, 128), jnp.int32),
        jax.ShapeDtypeStruct((32, 128), jnp.float32),
        jax.ShapeDtypeStruct((32, 128), jnp.float32),
        jax.ShapeDtypeStruct((32, 128), jnp.float32),
    ),
    scratch_shapes=[pltpu.VMEM((128, 128), jnp.float32)],
)


_RB = 512
_NBLK = N // _RB


def _mlp_body(x_ref, w_ref, b_ref, h_ref, scale_ref, shift_ref, s_ref, ss_ref):
    k = pl.program_id(0)
    h = jnp.dot(x_ref[...].astype(jnp.bfloat16), w_ref[...].astype(jnp.bfloat16),
                preferred_element_type=jnp.float32) + b_ref[...]
    h_ref[...] = h
    s = jnp.sum(h, axis=0, keepdims=True)
    ss = jnp.sum(h * h, axis=0, keepdims=True)

    @pl.when(k == 0)
    def _():
        s_ref[...] = s
        ss_ref[...] = ss

    @pl.when(k > 0)
    def _():
        s_ref[...] = s_ref[...] + s
        ss_ref[...] = ss_ref[...] + ss

    @pl.when(k == _NBLK - 1)
    def _():
        mean = s_ref[...] * (1.0 / N)
        var = ss_ref[...] * (1.0 / N) - mean * mean
        sc = lax.rsqrt(var + EPS)
        scale_ref[...] = sc
        shift_ref[...] = -mean * sc


_mlp_call = pl.pallas_call(
    _mlp_body,
    grid=(_NBLK,),
    in_specs=[
        pl.BlockSpec((_RB, IN_CH), lambda k: (k, 0)),
        pl.BlockSpec((IN_CH, OUT_CH), lambda k: (0, 0)),
        pl.BlockSpec((1, OUT_CH), lambda k: (0, 0)),
    ],
    out_specs=(
        pl.BlockSpec((_RB, OUT_CH), lambda k: (k, 0)),
        pl.BlockSpec((1, OUT_CH), lambda k: (0, 0)),
        pl.BlockSpec((1, OUT_CH), lambda k: (0, 0)),
    ),
    out_shape=(
        jax.ShapeDtypeStruct((N, OUT_CH), jnp.float32),
        jax.ShapeDtypeStruct((1, OUT_CH), jnp.float32),
        jax.ShapeDtypeStruct((1, OUT_CH), jnp.float32),
    ),
    scratch_shapes=[
        pltpu.VMEM((1, OUT_CH), jnp.float32),
        pltpu.VMEM((1, OUT_CH), jnp.float32),
    ],
)


_QB = 8


def _knn_body(qx_ref, qy_ref, qz_ref, px_ref, py_ref, pz_ref, nbr_ref):
    qx = qx_ref[...]
    qy = qy_ref[...]
    qz = qz_ref[...]
    px = px_ref[...]
    py = py_ref[...]
    pz = pz_ref[...]
    bf = jnp.bfloat16
    f32 = jnp.float32
    qbx = qx.astype(bf).astype(f32)
    qby = qy.astype(bf).astype(f32)
    qbz = qz.astype(bf).astype(f32)
    pbx = px.astype(bf).astype(f32)
    pby = py.astype(bf).astype(f32)
    pbz = pz.astype(bf).astype(f32)
    mm = qbx * pbx + qby * pby + qbz * pbz
    qn = qx * qx + qy * qy + qz * qz
    pn = px * px + py * py + pz * pz
    d = qn + pn - 2.0 * mm
    lin = lax.broadcasted_iota(jnp.int32, (_QB, N), 1)
    lane = lax.broadcasted_iota(jnp.int32, (_QB, 128), 1)

    m0 = jnp.min(d, axis=1, keepdims=True)
    p0 = jnp.min(jnp.where(d == m0, lin, BIGI), axis=1, keepdims=True)
    acc = jnp.broadcast_to(p0, (_QB, 128))
    d = jnp.where(lin == p0, jnp.inf, d)
    for j in range(1, KNN):
        mj = jnp.min(d, axis=1, keepdims=True)
        pj = jnp.min(jnp.where(d == mj, lin, BIGI), axis=1, keepdims=True)
        acc = jnp.where(lane == j, pj, acc)
        d = jnp.where(lin == pj, jnp.inf, d)
    nbr_ref[...] = acc


_knn_call = pl.pallas_call(
    _knn_body,
    grid=(M // _QB,),
    in_specs=[
        pl.BlockSpec((_QB, 1), lambda g: (g, 0)),
        pl.BlockSpec((_QB, 1), lambda g: (g, 0)),
        pl.BlockSpec((_QB, 1), lambda g: (g, 0)),
        pl.BlockSpec((1, N), lambda g: (0, 0)),
        pl.BlockSpec((1, N), lambda g: (0, 0)),
        pl.BlockSpec((1, N), lambda g: (0, 0)),
    ],
    out_specs=pl.BlockSpec((_QB, 128), lambda g: (g, 0)),
    out_shape=jax.ShapeDtypeStruct((M, 128), jnp.int32),
)


_NC = 2
_NS = 16
_NW = _NC * _NS
_QT = M // _NW
_GRP = 2


def _sc_body(h_hbm, idx_hbm, scale_hbm, shift_hbm, batch_hbm, clus_hbm,
             xout_hbm, sb_hbm,
             idx_v, rows_v, out_v, scale_v, shift_v, clus_v, sb_v, sem):
    wid = lax.axis_index("s") * _NC + lax.axis_index("c")
    qbase = wid * _QT
    pltpu.sync_copy(idx_hbm.at[pl.ds(qbase * KPAD, _QT * KPAD)], idx_v)
    pltpu.sync_copy(scale_hbm, scale_v)
    pltpu.sync_copy(shift_hbm, shift_v)
    pltpu.sync_copy(clus_hbm.at[pl.ds(qbase, _QT)], clus_v)
    pltpu.async_copy(batch_hbm.at[clus_v], sb_v, sem).wait()
    pltpu.sync_copy(sb_v, sb_hbm.at[pl.ds(qbase, _QT)])

    def group_body(g, carry):
        pltpu.async_copy(
            h_hbm.at[idx_v.at[pl.ds(g * _GRP * KPAD, _GRP * KPAD)]],
            rows_v, sem).wait()
        for q in range(_GRP):
            for ch in range(OUT_CH // 16):
                sl = pl.ds(ch * 16, 16)
                a = rows_v[q * KPAD, sl]
                for j in range(1, KPAD):
                    a = jnp.maximum(a, rows_v[q * KPAD + j, sl])
                out_v[q, sl] = jnp.maximum(a * scale_v[sl] + shift_v[sl], 0.0)
        pltpu.sync_copy(out_v, xout_hbm.at[pl.ds(qbase + g * _GRP, _GRP)])
        return carry

    lax.fori_loop(0, _QT // _GRP, group_body, 0)


@functools.lru_cache(maxsize=1)
def _get_sc_call():
    return pl.kernel(
        _sc_body,
        out_type=(
            jax.ShapeDtypeStruct((M, OUT_CH), jnp.float32),
            jax.ShapeDtypeStruct((M,), jnp.int32),
        ),
        mesh=plsc.VectorSubcoreMesh(core_axis_name="c", subcore_axis_name="s",
                                    num_cores=_NC, num_subcores=_NS),
        scratch_types=[
            pltpu.VMEM((_QT * KPAD,), jnp.int32),
            pltpu.VMEM((_GRP * KPAD, OUT_CH), jnp.float32),
            pltpu.VMEM((_GRP, OUT_CH), jnp.float32),
            pltpu.VMEM((OUT_CH,), jnp.float32),
            pltpu.VMEM((OUT_CH,), jnp.float32),
            pltpu.VMEM((_QT,), jnp.int32),
            pltpu.VMEM((_QT,), jnp.int32),
            pltpu.SemaphoreType.DMA,
        ],
    )


def kernel(x, pos, batch, W, b):
    px = pos[:, 0].reshape(128, 128)
    py = pos[:, 1].reshape(128, 128)
    pz = pos[:, 2].reshape(128, 128)

    idxq, qx, qy, qz = _fps_call(px, py, pz)

    h, scale, shift = _mlp_call(x, W, b.reshape(1, OUT_CH))

    nbr = _knn_call(
        qx.reshape(M, 1), qy.reshape(M, 1), qz.reshape(M, 1),
        px.reshape(1, N), py.reshape(1, N), pz.reshape(1, N),
    )

    flat_idx = nbr[:, :KPAD].reshape(-1)
    x_out, sub_batch = _get_sc_call()(
        h, flat_idx, scale.reshape(OUT_CH), shift.reshape(OUT_CH),
        batch, idxq.reshape(M),
    )
    sub_pos = jnp.concatenate(
        [qx.reshape(M, 1), qy.reshape(M, 1), qz.reshape(M, 1)], axis=1)
    return (x_out, sub_pos, sub_batch)


# speedup vs baseline: 11.5762x; 1.6643x over previous
import functools

import jax
import jax.numpy as jnp
from jax import lax
from jax.experimental import pallas as pl
from jax.experimental.pallas import tpu as pltpu
from jax.experimental.pallas import tpu_sc as plsc

N = 16384
M = 4096
KNN = 17
KPAD = 20
IN_CH = 256
OUT_CH = 512
EPS = 1e-5
BIGI = 2 ** 30


def _fps_body(px_ref, py_ref, pz_ref, idx_ref, qx_ref, qy_ref, qz_ref, dist_ref):
    px = px_ref[...]
    py = py_ref[...]
    pz = pz_ref[...]
    lin = (lax.broadcasted_iota(jnp.int32, (128, 128), 0) * 128
           + lax.broadcasted_iota(jnp.int32, (128, 128), 1))
    olin = (lax.broadcasted_iota(jnp.int32, (32, 128), 0) * 128
            + lax.broadcasted_iota(jnp.int32, (32, 128), 1))

    first = lin == 0
    x0 = jnp.sum(jnp.where(first, px, 0.0))
    y0 = jnp.sum(jnp.where(first, py, 0.0))
    z0 = jnp.sum(jnp.where(first, pz, 0.0))
    w0 = olin == 0
    idx_ref[...] = jnp.zeros((32, 128), jnp.int32)
    qx_ref[...] = jnp.where(w0, x0, 0.0)
    qy_ref[...] = jnp.where(w0, y0, 0.0)
    qz_ref[...] = jnp.where(w0, z0, 0.0)
    dist_ref[...] = jnp.full((128, 128), jnp.inf, jnp.float32)

    def body(i, carry):
        lx, ly, lz = carry
        dx = px - lx
        dy = py - ly
        dz = pz - lz
        d = dx * dx + dy * dy + dz * dz
        dist = jnp.minimum(dist_ref[...], d)
        dist_ref[...] = dist
        m = jnp.max(dist)
        pidx = jnp.min(jnp.where(dist == m, lin, BIGI))
        sel = lin == pidx
        nx = jnp.sum(jnp.where(sel, px, 0.0))
        ny = jnp.sum(jnp.where(sel, py, 0.0))
        nz = jnp.sum(jnp.where(sel, pz, 0.0))
        w = olin == i
        idx_ref[...] = jnp.where(w, pidx, idx_ref[...])
        qx_ref[...] = jnp.where(w, nx, qx_ref[...])
        qy_ref[...] = jnp.where(w, ny, qy_ref[...])
        qz_ref[...] = jnp.where(w, nz, qz_ref[...])
        return (nx, ny, nz)

    lax.fori_loop(1, M, body, (x0, y0, z0))


_fps_call = pl.pallas_call(
    _fps_body,
    out_shape=(
        jax.ShapeDtypeStruct((32, 128), jnp.int32),
        jax.ShapeDtypeStruct((32, 128), jnp.float32),
        jax.ShapeDtypeStruct((32, 128), jnp.float32),
        jax.ShapeDtypeStruct((32, 128), jnp.float32),
    ),
    scratch_shapes=[pltpu.VMEM((128, 128), jnp.float32)],
)


_RB = 512
_NBLK = N // _RB


def _mlp_body(x_ref, w_ref, b_ref, h_ref, scale_ref, shift_ref, s_ref, ss_ref):
    k = pl.program_id(0)
    h = jnp.dot(x_ref[...].astype(jnp.bfloat16), w_ref[...].astype(jnp.bfloat16),
                preferred_element_type=jnp.float32) + b_ref[...]
    h_ref[...] = h
    s = jnp.sum(h, axis=0, keepdims=True)
    ss = jnp.sum(h * h, axis=0, keepdims=True)

    @pl.when(k == 0)
    def _():
        s_ref[...] = s
        ss_ref[...] = ss

    @pl.when(k > 0)
    def _():
        s_ref[...] = s_ref[...] + s
        ss_ref[...] = ss_ref[...] + ss

    @pl.when(k == _NBLK - 1)
    def _():
        mean = s_ref[...] * (1.0 / N)
        var = ss_ref[...] * (1.0 / N) - mean * mean
        sc = lax.rsqrt(var + EPS)
        scale_ref[...] = sc
        shift_ref[...] = -mean * sc


_mlp_call = pl.pallas_call(
    _mlp_body,
    grid=(_NBLK,),
    in_specs=[
        pl.BlockSpec((_RB, IN_CH), lambda k: (k, 0)),
        pl.BlockSpec((IN_CH, OUT_CH), lambda k: (0, 0)),
        pl.BlockSpec((1, OUT_CH), lambda k: (0, 0)),
    ],
    out_specs=(
        pl.BlockSpec((_RB, OUT_CH), lambda k: (k, 0)),
        pl.BlockSpec((1, OUT_CH), lambda k: (0, 0)),
        pl.BlockSpec((1, OUT_CH), lambda k: (0, 0)),
    ),
    out_shape=(
        jax.ShapeDtypeStruct((N, OUT_CH), jnp.float32),
        jax.ShapeDtypeStruct((1, OUT_CH), jnp.float32),
        jax.ShapeDtypeStruct((1, OUT_CH), jnp.float32),
    ),
    scratch_shapes=[
        pltpu.VMEM((1, OUT_CH), jnp.float32),
        pltpu.VMEM((1, OUT_CH), jnp.float32),
    ],
)


_QB = 64


def _knn_body(qx_ref, qy_ref, qz_ref, px_ref, py_ref, pz_ref, nbr_ref):
    qx = qx_ref[...]
    qy = qy_ref[...]
    qz = qz_ref[...]
    px = px_ref[...]
    py = py_ref[...]
    pz = pz_ref[...]
    bf = jnp.bfloat16
    f32 = jnp.float32
    qbx = qx.astype(bf).astype(f32)
    qby = qy.astype(bf).astype(f32)
    qbz = qz.astype(bf).astype(f32)
    pbx = px.astype(bf).astype(f32)
    pby = py.astype(bf).astype(f32)
    pbz = pz.astype(bf).astype(f32)
    mm = qbx * pbx + qby * pby + qbz * pbz
    qn = qx * qx + qy * qy + qz * qz
    pn = px * px + py * py + pz * pz
    d = qn + pn - 2.0 * mm
    lin = lax.broadcasted_iota(jnp.int32, (_QB, N), 1)
    lane = lax.broadcasted_iota(jnp.int32, (_QB, 128), 1)

    m0 = jnp.min(d, axis=1, keepdims=True)
    p0 = jnp.min(jnp.where(d == m0, lin, BIGI), axis=1, keepdims=True)
    acc = jnp.broadcast_to(p0, (_QB, 128))
    d = jnp.where(lin == p0, jnp.inf, d)
    for j in range(1, KNN):
        mj = jnp.min(d, axis=1, keepdims=True)
        pj = jnp.min(jnp.where(d == mj, lin, BIGI), axis=1, keepdims=True)
        acc = jnp.where(lane == j, pj, acc)
        d = jnp.where(lin == pj, jnp.inf, d)
    nbr_ref[...] = acc


_knn_call = pl.pallas_call(
    _knn_body,
    grid=(M // _QB,),
    in_specs=[
        pl.BlockSpec((_QB, 1), lambda g: (g, 0)),
        pl.BlockSpec((_QB, 1), lambda g: (g, 0)),
        pl.BlockSpec((_QB, 1), lambda g: (g, 0)),
        pl.BlockSpec((1, N), lambda g: (0, 0)),
        pl.BlockSpec((1, N), lambda g: (0, 0)),
        pl.BlockSpec((1, N), lambda g: (0, 0)),
    ],
    out_specs=pl.BlockSpec((_QB, 128), lambda g: (g, 0)),
    out_shape=jax.ShapeDtypeStruct((M, 128), jnp.int32),
)


_NC = 2
_NS = 16
_NW = _NC * _NS
_QT = M // _NW
_GRP = 2


def _sc_body(h_hbm, idx_hbm, scale_hbm, shift_hbm, batch_hbm, clus_hbm,
             xout_hbm, sb_hbm,
             idx_v, rows_v, out_v, scale_v, shift_v, clus_v, sb_v, sem):
    wid = lax.axis_index("s") * _NC + lax.axis_index("c")
    qbase = wid * _QT
    pltpu.sync_copy(idx_hbm.at[pl.ds(qbase * KPAD, _QT * KPAD)], idx_v)
    pltpu.sync_copy(scale_hbm, scale_v)
    pltpu.sync_copy(shift_hbm, shift_v)
    pltpu.sync_copy(clus_hbm.at[pl.ds(qbase, _QT)], clus_v)
    pltpu.async_copy(batch_hbm.at[clus_v], sb_v, sem).wait()
    pltpu.sync_copy(sb_v, sb_hbm.at[pl.ds(qbase, _QT)])

    def group_body(g, carry):
        pltpu.async_copy(
            h_hbm.at[idx_v.at[pl.ds(g * _GRP * KPAD, _GRP * KPAD)]],
            rows_v, sem).wait()
        for q in range(_GRP):
            for ch in range(OUT_CH // 16):
                sl = pl.ds(ch * 16, 16)
                a = rows_v[q * KPAD, sl]
                for j in range(1, KPAD):
                    a = jnp.maximum(a, rows_v[q * KPAD + j, sl])
                out_v[q, sl] = jnp.maximum(a * scale_v[sl] + shift_v[sl], 0.0)
        pltpu.sync_copy(out_v, xout_hbm.at[pl.ds(qbase + g * _GRP, _GRP)])
        return carry

    lax.fori_loop(0, _QT // _GRP, group_body, 0)


@functools.lru_cache(maxsize=1)
def _get_sc_call():
    return pl.kernel(
        _sc_body,
        out_type=(
            jax.ShapeDtypeStruct((M, OUT_CH), jnp.float32),
            jax.ShapeDtypeStruct((M,), jnp.int32),
        ),
        mesh=plsc.VectorSubcoreMesh(core_axis_name="c", subcore_axis_name="s",
                                    num_cores=_NC, num_subcores=_NS),
        scratch_types=[
            pltpu.VMEM((_QT * KPAD,), jnp.int32),
            pltpu.VMEM((_GRP * KPAD, OUT_CH), jnp.float32),
            pltpu.VMEM((_GRP, OUT_CH), jnp.float32),
            pltpu.VMEM((OUT_CH,), jnp.float32),
            pltpu.VMEM((OUT_CH,), jnp.float32),
            pltpu.VMEM((_QT,), jnp.int32),
            pltpu.VMEM((_QT,), jnp.int32),
            pltpu.SemaphoreType.DMA,
        ],
    )


def kernel(x, pos, batch, W, b):
    px = pos[:, 0].reshape(128, 128)
    py = pos[:, 1].reshape(128, 128)
    pz = pos[:, 2].reshape(128, 128)

    idxq, qx, qy, qz = _fps_call(px, py, pz)

    h, scale, shift = _mlp_call(x, W, b.reshape(1, OUT_CH))

    nbr = _knn_call(
        qx.reshape(M, 1), qy.reshape(M, 1), qz.reshape(M, 1),
        px.reshape(1, N), py.reshape(1, N), pz.reshape(1, N),
    )

    flat_idx = nbr[:, :KPAD].reshape(-1)
    x_out, sub_batch = _get_sc_call()(
        h, flat_idx, scale.reshape(OUT_CH), shift.reshape(OUT_CH),
        batch, idxq.reshape(M),
    )
    sub_pos = jnp.concatenate(
        [qx.reshape(M, 1), qy.reshape(M, 1), qz.reshape(M, 1)], axis=1)
    return (x_out, sub_pos, sub_batch)


# speedup vs baseline: 12.8716x; 1.1119x over previous
import functools

import jax
import jax.numpy as jnp
from jax import lax
from jax.experimental import pallas as pl
from jax.experimental.pallas import tpu as pltpu
from jax.experimental.pallas import tpu_sc as plsc

N = 16384
M = 4096
KNN = 17
KPAD = 20
IN_CH = 256
OUT_CH = 512
EPS = 1e-5
BIGI = 2 ** 30


def _fps_body(px_ref, py_ref, pz_ref, px1_ref, py1_ref, pz1_ref,
              idx_ref, qx_ref, qy_ref, qz_ref, dist_ref):
    px = px_ref[...]
    py = py_ref[...]
    pz = pz_ref[...]
    lin = (lax.broadcasted_iota(jnp.int32, (128, 128), 0) * 128
           + lax.broadcasted_iota(jnp.int32, (128, 128), 1))
    olin = (lax.broadcasted_iota(jnp.int32, (32, 128), 0) * 128
            + lax.broadcasted_iota(jnp.int32, (32, 128), 1))

    first = lin == 0
    x0 = jnp.sum(jnp.where(first, px, 0.0))
    y0 = jnp.sum(jnp.where(first, py, 0.0))
    z0 = jnp.sum(jnp.where(first, pz, 0.0))
    w0 = olin == 0
    idx_ref[...] = jnp.zeros((32, 128), jnp.int32)
    qx_ref[...] = jnp.where(w0, x0, 0.0)
    qy_ref[...] = jnp.where(w0, y0, 0.0)
    qz_ref[...] = jnp.where(w0, z0, 0.0)
    dist_ref[...] = jnp.full((128, 128), jnp.inf, jnp.float32)

    def body(i, carry):
        lx, ly, lz = carry
        dx = px - lx
        dy = py - ly
        dz = pz - lz
        d = dx * dx + dy * dy + dz * dz
        dist = jnp.minimum(dist_ref[...], d)
        dist_ref[...] = dist
        m = jnp.max(dist)
        pidx = jnp.min(jnp.where(dist == m, lin, BIGI))
        nx = px1_ref[pidx, 0]
        ny = py1_ref[pidx, 0]
        nz = pz1_ref[pidx, 0]
        w = olin == i
        idx_ref[...] = jnp.where(w, pidx, idx_ref[...])
        qx_ref[...] = jnp.where(w, nx, qx_ref[...])
        qy_ref[...] = jnp.where(w, ny, qy_ref[...])
        qz_ref[...] = jnp.where(w, nz, qz_ref[...])
        return (nx, ny, nz)

    lax.fori_loop(1, M, body, (x0, y0, z0))


_fps_call = pl.pallas_call(
    _fps_body,
    out_shape=(
        jax.ShapeDtypeStruct((32, 128), jnp.int32),
        jax.ShapeDtypeStruct((32, 128), jnp.float32),
        jax.ShapeDtypeStruct((32, 128), jnp.float32),
        jax.ShapeDtypeStruct((32, 128), jnp.float32),
    ),
    scratch_shapes=[pltpu.VMEM((128, 128), jnp.float32)],
)


_RB = 512
_NBLK = N // _RB


def _mlp_body(x_ref, w_ref, b_ref, h_ref, scale_ref, shift_ref, s_ref, ss_ref):
    k = pl.program_id(0)
    h = jnp.dot(x_ref[...].astype(jnp.bfloat16), w_ref[...].astype(jnp.bfloat16),
                preferred_element_type=jnp.float32) + b_ref[...]
    h_ref[...] = h
    s = jnp.sum(h, axis=0, keepdims=True)
    ss = jnp.sum(h * h, axis=0, keepdims=True)

    @pl.when(k == 0)
    def _():
        s_ref[...] = s
        ss_ref[...] = ss

    @pl.when(k > 0)
    def _():
        s_ref[...] = s_ref[...] + s
        ss_ref[...] = ss_ref[...] + ss

    @pl.when(k == _NBLK - 1)
    def _():
        mean = s_ref[...] * (1.0 / N)
        var = ss_ref[...] * (1.0 / N) - mean * mean
        sc = lax.rsqrt(var + EPS)
        scale_ref[...] = sc
        shift_ref[...] = -mean * sc


_mlp_call = pl.pallas_call(
    _mlp_body,
    grid=(_NBLK,),
    in_specs=[
        pl.BlockSpec((_RB, IN_CH), lambda k: (k, 0)),
        pl.BlockSpec((IN_CH, OUT_CH), lambda k: (0, 0)),
        pl.BlockSpec((1, OUT_CH), lambda k: (0, 0)),
    ],
    out_specs=(
        pl.BlockSpec((_RB, OUT_CH), lambda k: (k, 0)),
        pl.BlockSpec((1, OUT_CH), lambda k: (0, 0)),
        pl.BlockSpec((1, OUT_CH), lambda k: (0, 0)),
    ),
    out_shape=(
        jax.ShapeDtypeStruct((N, OUT_CH), jnp.float32),
        jax.ShapeDtypeStruct((1, OUT_CH), jnp.float32),
        jax.ShapeDtypeStruct((1, OUT_CH), jnp.float32),
    ),
    scratch_shapes=[
        pltpu.VMEM((1, OUT_CH), jnp.float32),
        pltpu.VMEM((1, OUT_CH), jnp.float32),
    ],
)


_QB = 64


def _knn_body(qx_ref, qy_ref, qz_ref, px_ref, py_ref, pz_ref, nbr_ref):
    qx = qx_ref[...]
    qy = qy_ref[...]
    qz = qz_ref[...]
    px = px_ref[...]
    py = py_ref[...]
    pz = pz_ref[...]
    bf = jnp.bfloat16
    f32 = jnp.float32
    qbx = qx.astype(bf).astype(f32)
    qby = qy.astype(bf).astype(f32)
    qbz = qz.astype(bf).astype(f32)
    pbx = px.astype(bf).astype(f32)
    pby = py.astype(bf).astype(f32)
    pbz = pz.astype(bf).astype(f32)
    mm = qbx * pbx + qby * pby + qbz * pbz
    qn = qx * qx + qy * qy + qz * qz
    pn = px * px + py * py + pz * pz
    d = qn + pn - 2.0 * mm
    lin = lax.broadcasted_iota(jnp.int32, (_QB, N), 1)
    lane = lax.broadcasted_iota(jnp.int32, (_QB, 128), 1)

    m0 = jnp.min(d, axis=1, keepdims=True)
    p0 = jnp.min(jnp.where(d == m0, lin, BIGI), axis=1, keepdims=True)
    acc = jnp.broadcast_to(p0, (_QB, 128))
    d = jnp.where(lin == p0, jnp.inf, d)
    for j in range(1, KNN):
        mj = jnp.min(d, axis=1, keepdims=True)
        pj = jnp.min(jnp.where(d == mj, lin, BIGI), axis=1, keepdims=True)
        acc = jnp.where(lane == j, pj, acc)
        d = jnp.where(lin == pj, jnp.inf, d)
    nbr_ref[...] = acc


_knn_call = pl.pallas_call(
    _knn_body,
    grid=(M // _QB,),
    in_specs=[
        pl.BlockSpec((_QB, 1), lambda g: (g, 0)),
        pl.BlockSpec((_QB, 1), lambda g: (g, 0)),
        pl.BlockSpec((_QB, 1), lambda g: (g, 0)),
        pl.BlockSpec((1, N), lambda g: (0, 0)),
        pl.BlockSpec((1, N), lambda g: (0, 0)),
        pl.BlockSpec((1, N), lambda g: (0, 0)),
    ],
    out_specs=pl.BlockSpec((_QB, 128), lambda g: (g, 0)),
    out_shape=jax.ShapeDtypeStruct((M, 128), jnp.int32),
)


_NC = 2
_NS = 16
_NW = _NC * _NS
_QT = M // _NW
_GRP = 2


def _sc_body(h_hbm, idx_hbm, scale_hbm, shift_hbm, batch_hbm, clus_hbm,
             xout_hbm, sb_hbm,
             idx_v, rows_v, out_v, scale_v, shift_v, clus_v, sb_v, sem):
    wid = lax.axis_index("s") * _NC + lax.axis_index("c")
    qbase = wid * _QT
    pltpu.sync_copy(idx_hbm.at[pl.ds(qbase * KPAD, _QT * KPAD)], idx_v)
    pltpu.sync_copy(scale_hbm, scale_v)
    pltpu.sync_copy(shift_hbm, shift_v)
    pltpu.sync_copy(clus_hbm.at[pl.ds(qbase, _QT)], clus_v)
    pltpu.async_copy(batch_hbm.at[clus_v], sb_v, sem).wait()
    pltpu.sync_copy(sb_v, sb_hbm.at[pl.ds(qbase, _QT)])

    def group_body(g, carry):
        pltpu.async_copy(
            h_hbm.at[idx_v.at[pl.ds(g * _GRP * KPAD, _GRP * KPAD)]],
            rows_v, sem).wait()
        for q in range(_GRP):
            for ch in range(OUT_CH // 16):
                sl = pl.ds(ch * 16, 16)
                a = rows_v[q * KPAD, sl]
                for j in range(1, KPAD):
                    a = jnp.maximum(a, rows_v[q * KPAD + j, sl])
                out_v[q, sl] = jnp.maximum(a * scale_v[sl] + shift_v[sl], 0.0)
        pltpu.sync_copy(out_v, xout_hbm.at[pl.ds(qbase + g * _GRP, _GRP)])
        return carry

    lax.fori_loop(0, _QT // _GRP, group_body, 0)


@functools.lru_cache(maxsize=1)
def _get_sc_call():
    return pl.kernel(
        _sc_body,
        out_type=(
            jax.ShapeDtypeStruct((M, OUT_CH), jnp.float32),
            jax.ShapeDtypeStruct((M,), jnp.int32),
        ),
        mesh=plsc.VectorSubcoreMesh(core_axis_name="c", subcore_axis_name="s",
                                    num_cores=_NC, num_subcores=_NS),
        scratch_types=[
            pltpu.VMEM((_QT * KPAD,), jnp.int32),
            pltpu.VMEM((_GRP * KPAD, OUT_CH), jnp.float32),
            pltpu.VMEM((_GRP, OUT_CH), jnp.float32),
            pltpu.VMEM((OUT_CH,), jnp.float32),
            pltpu.VMEM((OUT_CH,), jnp.float32),
            pltpu.VMEM((_QT,), jnp.int32),
            pltpu.VMEM((_QT,), jnp.int32),
            pltpu.SemaphoreType.DMA,
        ],
    )


def kernel(x, pos, batch, W, b):
    px = pos[:, 0].reshape(128, 128)
    py = pos[:, 1].reshape(128, 128)
    pz = pos[:, 2].reshape(128, 128)

    idxq, qx, qy, qz = _fps_call(px, py, pz,
                                 pos[:, 0].reshape(N, 1),
                                 pos[:, 1].reshape(N, 1),
                                 pos[:, 2].reshape(N, 1))

    h, scale, shift = _mlp_call(x, W, b.reshape(1, OUT_CH))

    nbr = _knn_call(
        qx.reshape(M, 1), qy.reshape(M, 1), qz.reshape(M, 1),
        px.reshape(1, N), py.reshape(1, N), pz.reshape(1, N),
    )

    flat_idx = nbr[:, :KPAD].reshape(-1)
    x_out, sub_batch = _get_sc_call()(
        h, flat_idx, scale.reshape(OUT_CH), shift.reshape(OUT_CH),
        batch, idxq.reshape(M),
    )
    sub_pos = jnp.concatenate(
        [qx.reshape(M, 1), qy.reshape(M, 1), qz.reshape(M, 1)], axis=1)
    return (x_out, sub_pos, sub_batch)
